# baseline (speedup 1.0000x reference)
import functools

import jax
import jax.numpy as jnp
from jax import lax
from jax.experimental import pallas as pl
from jax.experimental.pallas import tpu as pltpu
from jax.experimental.pallas import tpu_sc as plsc

NUM_CODES = 100000
EMB = 32
BATCH = 16384
NT = 4
NC = 2
NS = 16
NW = NC * NS
B_PER_W = BATCH // NW
CH = 128
NCHUNK = B_PER_W // CH
D = NT * EMB


TBLK = 512
NBLK = -(-NUM_CODES // TBLK)


@functools.cache
def _build_tc_relayout():
    def body(w0_ref, w1_ref, w2_ref, w3_ref, out_ref):
        for t, w in enumerate((w0_ref, w1_ref, w2_ref, w3_ref)):
            out_ref[:, t * EMB:(t + 1) * EMB] = w[...].T

    in_spec = pl.BlockSpec((EMB, TBLK), lambda i: (0, i))
    return pl.pallas_call(
        body,
        grid=(NBLK,),
        in_specs=[in_spec] * NT,
        out_specs=pl.BlockSpec((TBLK, D), lambda i: (i, 0)),
        out_shape=jax.ShapeDtypeStruct((NUM_CODES, D), jnp.float32),
    )


@functools.cache
def _build():
    mesh = plsc.VectorSubcoreMesh(core_axis_name="c", subcore_axis_name="s")

    scratch = (
        [pltpu.VMEM((CH,), jnp.int32) for _ in range(NCHUNK)]
        + [pltpu.VMEM((CH, D), jnp.float32) for _ in range(NCHUNK)]
        + [pltpu.SemaphoreType.DMA, pltpu.SemaphoreType.DMA]
    )

    @functools.partial(
        pl.kernel,
        mesh=mesh,
        out_type=jax.ShapeDtypeStruct((BATCH, D), jnp.float32),
        scratch_types=scratch,
    )
    def sc_gather(ids_hbm, wcat_hbm, out_hbm, *scratch_refs):
        idx_v = scratch_refs[:NCHUNK]
        rows_v = scratch_refs[NCHUNK:2 * NCHUNK]
        gsem, osem = scratch_refs[-2:]
        wid = lax.axis_index("s") * NC + lax.axis_index("c")
        base = wid * B_PER_W

        for j in range(NCHUNK):
            pltpu.sync_copy(ids_hbm.at[pl.ds(base + j * CH, CH)], idx_v[j])

        gathers = [
            pltpu.async_copy(wcat_hbm.at[idx_v[j]], rows_v[j], gsem)
            for j in range(NCHUNK)
        ]
        for g in gathers:
            g.wait()

        writes = [
            pltpu.async_copy(rows_v[j], out_hbm.at[pl.ds(base + j * CH, CH)],
                             osem)
            for j in range(NCHUNK)
        ]
        for w in writes:
            w.wait()

    return sc_gather


def kernel(code_ids, W0, W1, W2, W3):
    ids = code_ids.astype(jnp.int32)
    wcat = _build_tc_relayout()(W0.T, W1.T, W2.T, W3.T)
    return _build()(ids, wcat)


# speedup vs baseline: 1.8362x; 1.8362x over previous
import functools

import jax
import jax.numpy as jnp
from jax import lax
from jax.experimental import pallas as pl
from jax.experimental.pallas import tpu as pltpu
from jax.experimental.pallas import tpu_sc as plsc

NUM_CODES = 100000
EMB = 32
BATCH = 16384
NT = 4
NC = 2
NS = 16
NW = NC * NS
B_PER_W = BATCH // NW
CH = 128
NCHUNK = B_PER_W // CH
D = NT * EMB


TBLK = 1024
NBLK = -(-NUM_CODES // TBLK)


@functools.cache
def _build_tc_relayout():
    def body(w0_ref, w1_ref, w2_ref, w3_ref, out_ref):
        stacked = jnp.concatenate(
            [w[...] for w in (w0_ref, w1_ref, w2_ref, w3_ref)], axis=0
        )
        out_ref[...] = stacked.T

    in_spec = pl.BlockSpec((EMB, TBLK), lambda i: (0, i))
    return pl.pallas_call(
        body,
        grid=(NBLK,),
        in_specs=[in_spec] * NT,
        out_specs=pl.BlockSpec((TBLK, D), lambda i: (i, 0)),
        out_shape=jax.ShapeDtypeStruct((NUM_CODES, D), jnp.float32),
        compiler_params=pltpu.CompilerParams(
            fuse_transposed_lhs_in_matmul=True,
        ),
    )


@functools.cache
def _build():
    mesh = plsc.VectorSubcoreMesh(core_axis_name="c", subcore_axis_name="s")

    scratch = (
        [pltpu.VMEM((CH,), jnp.int32) for _ in range(NCHUNK)]
        + [pltpu.VMEM((CH, D), jnp.float32) for _ in range(NCHUNK)]
        + [pltpu.SemaphoreType.DMA, pltpu.SemaphoreType.DMA]
    )

    @functools.partial(
        pl.kernel,
        mesh=mesh,
        out_type=jax.ShapeDtypeStruct((BATCH, D), jnp.float32),
        scratch_types=scratch,
    )
    def sc_gather(ids_hbm, wcat_hbm, out_hbm, *scratch_refs):
        idx_v = scratch_refs[:NCHUNK]
        rows_v = scratch_refs[NCHUNK:2 * NCHUNK]
        gsem, osem = scratch_refs[-2:]
        wid = lax.axis_index("s") * NC + lax.axis_index("c")
        base = wid * B_PER_W

        for j in range(NCHUNK):
            pltpu.sync_copy(ids_hbm.at[pl.ds(base + j * CH, CH)], idx_v[j])

        gathers = [
            pltpu.async_copy(wcat_hbm.at[idx_v[j]], rows_v[j], gsem)
            for j in range(NCHUNK)
        ]
        for g in gathers:
            g.wait()

        writes = [
            pltpu.async_copy(rows_v[j], out_hbm.at[pl.ds(base + j * CH, CH)],
                             osem)
            for j in range(NCHUNK)
        ]
        for w in writes:
            w.wait()

    return sc_gather


def kernel(code_ids, W0, W1, W2, W3):
    ids = code_ids.astype(jnp.int32)
    wcat = _build_tc_relayout()(W0.T, W1.T, W2.T, W3.T)
    return _build()(ids, wcat)
